# baseline (speedup 1.0000x reference)
import functools

import jax
import jax.numpy as jnp
from jax import lax
from jax.experimental import pallas as pl
from jax.experimental.pallas import tpu as pltpu
from jax.experimental.pallas import tpu_sc as plsc

VOCAB = 100000
EMBD = 128
HIDDEN = 128

_INFO = plsc.get_sparse_core_info()
_NC, _NS = _INFO.num_cores, _INFO.num_subcores
_NW = _NC * _NS

_CHUNK = 128


def _proj_kernel(wv_ref, w_ref, out_ref):
    out_ref[...] = lax.dot_general(
        wv_ref[...], w_ref[...],
        dimension_numbers=(((1,), (1,)), ((), ())),
        preferred_element_type=jnp.float32,
    )


def _project_table(word_vectors, W_proj):
    rows_per_block = 1000
    grid = VOCAB // rows_per_block
    return pl.pallas_call(
        _proj_kernel,
        grid=(grid,),
        in_specs=[
            pl.BlockSpec((rows_per_block, EMBD), lambda i: (i, 0)),
            pl.BlockSpec((HIDDEN, EMBD), lambda i: (0, 0)),
        ],
        out_specs=pl.BlockSpec((rows_per_block, HIDDEN), lambda i: (i, 0)),
        out_shape=jax.ShapeDtypeStruct((VOCAB, HIDDEN), jnp.float32),
    )(word_vectors, W_proj)


_NBUF = 4


def _make_gather(total, d):
    assert total % (_NW * _CHUNK) == 0
    per_worker = total // _NW
    n_groups = per_worker // _CHUNK
    assert n_groups % _NBUF == 0 and n_groups >= 2 * _NBUF
    mesh = plsc.VectorSubcoreMesh(core_axis_name="c", subcore_axis_name="s")

    @functools.partial(
        pl.kernel,
        mesh=mesh,
        out_type=jax.ShapeDtypeStruct((total, d), jnp.float32),
        scratch_types=[
            pltpu.VMEM((n_groups, _CHUNK), jnp.int32),
        ]
        + [pltpu.VMEM((_CHUNK, d), jnp.float32)] * _NBUF
        + [pltpu.SemaphoreType.DMA] * (2 * _NBUF),
    )
    def gather(table_hbm, idx2d_hbm, out_hbm, idx_all,
               b0, b1, b2, b3, g0, g1, g2, g3, s0, s1, s2, s3):
        wid = lax.axis_index("s") * _NC + lax.axis_index("c")
        base = wid * per_worker
        bufs = (b0, b1, b2, b3)
        gsems = (g0, g1, g2, g3)
        ssems = (s0, s1, s2, s3)

        pltpu.sync_copy(idx2d_hbm.at[pl.ds(wid * n_groups, n_groups)], idx_all)

        def fire(grp, slot):
            pltpu.async_copy(
                table_hbm.at[idx_all.at[grp]], bufs[slot], gsems[slot])

        def drain_gather(slot):
            pltpu.make_async_copy(
                out_hbm.at[pl.ds(0, _CHUNK)], bufs[slot], gsems[slot]).wait()

        def fire_store(grp, slot):
            pltpu.async_copy(
                bufs[slot], out_hbm.at[pl.ds(base + grp * _CHUNK, _CHUNK)],
                ssems[slot])

        def drain_store(slot):
            pltpu.make_async_copy(
                bufs[slot], out_hbm.at[pl.ds(0, _CHUNK)], ssems[slot]).wait()

        fire(0, 0)
        fire(1, 1)

        @pl.loop(0, n_groups, step=_NBUF)
        def _group(g):
            for p in range(_NBUF):
                grp = g + p
                q = (p + 2) % _NBUF
                drain_gather(p)
                fire_store(grp, p)

                @pl.when(grp >= 2)
                def _wait_prev_store():
                    drain_store(q)

                @pl.when(grp + 2 < n_groups)
                def _refill():
                    fire(grp + 2, q)

        drain_store((n_groups - 2) % _NBUF)
        drain_store((n_groups - 1) % _NBUF)

    return gather


@jax.jit
def kernel(x, word_vectors, W_proj):
    b, s = x.shape
    proj_table = _project_table(word_vectors, W_proj)
    flat_idx = x.reshape(-1).astype(jnp.int32)
    idx2d = flat_idx.reshape(-1, _CHUNK)
    out = _make_gather(b * s, HIDDEN)(proj_table, idx2d)
    return out.reshape(b, s, HIDDEN)


# speedup vs baseline: 1.1198x; 1.1198x over previous
import functools

import jax
import jax.numpy as jnp
from jax import lax
from jax.experimental import pallas as pl
from jax.experimental.pallas import tpu as pltpu
from jax.experimental.pallas import tpu_sc as plsc

VOCAB = 100000
EMBD = 128
HIDDEN = 128

_INFO = plsc.get_sparse_core_info()
_NC, _NS = _INFO.num_cores, _INFO.num_subcores
_NW = _NC * _NS

_CHUNK = 128


def _proj_kernel(wv_ref, w_ref, out_ref):
    out_ref[...] = lax.dot_general(
        wv_ref[...], w_ref[...],
        dimension_numbers=(((1,), (1,)), ((), ())),
        preferred_element_type=jnp.float32,
    )


def _project_table(word_vectors, W_proj):
    rows_per_block = 5000
    grid = VOCAB // rows_per_block
    return pl.pallas_call(
        _proj_kernel,
        grid=(grid,),
        in_specs=[
            pl.BlockSpec((rows_per_block, EMBD), lambda i: (i, 0)),
            pl.BlockSpec((HIDDEN, EMBD), lambda i: (0, 0)),
        ],
        out_specs=pl.BlockSpec((rows_per_block, HIDDEN), lambda i: (i, 0)),
        out_shape=jax.ShapeDtypeStruct((VOCAB, HIDDEN), jnp.float32),
    )(word_vectors, W_proj)


_NBUF = 4


def _make_gather(total, d):
    assert total % (_NW * _CHUNK) == 0
    per_worker = total // _NW
    n_groups = per_worker // _CHUNK
    assert n_groups % _NBUF == 0 and n_groups >= 2 * _NBUF
    mesh = plsc.VectorSubcoreMesh(core_axis_name="c", subcore_axis_name="s")

    @functools.partial(
        pl.kernel,
        mesh=mesh,
        out_type=jax.ShapeDtypeStruct((total, d), jnp.float32),
        scratch_types=[
            pltpu.VMEM((n_groups, _CHUNK), jnp.int32),
        ]
        + [pltpu.VMEM((_CHUNK, d), jnp.float32)] * _NBUF
        + [pltpu.SemaphoreType.DMA] * (2 * _NBUF),
    )
    def gather(table_hbm, idx2d_hbm, out_hbm, idx_all,
               b0, b1, b2, b3, g0, g1, g2, g3, s0, s1, s2, s3):
        wid = lax.axis_index("s") * _NC + lax.axis_index("c")
        base = wid * per_worker
        bufs = (b0, b1, b2, b3)
        gsems = (g0, g1, g2, g3)
        ssems = (s0, s1, s2, s3)

        pltpu.sync_copy(idx2d_hbm.at[pl.ds(wid * n_groups, n_groups)], idx_all)

        def fire(grp, slot):
            pltpu.async_copy(
                table_hbm.at[idx_all.at[grp]], bufs[slot], gsems[slot])

        def drain_gather(slot):
            pltpu.make_async_copy(
                out_hbm.at[pl.ds(0, _CHUNK)], bufs[slot], gsems[slot]).wait()

        def fire_store(grp, slot):
            pltpu.async_copy(
                bufs[slot], out_hbm.at[pl.ds(base + grp * _CHUNK, _CHUNK)],
                ssems[slot])

        def drain_store(slot):
            pltpu.make_async_copy(
                bufs[slot], out_hbm.at[pl.ds(0, _CHUNK)], ssems[slot]).wait()

        fire(0, 0)
        fire(1, 1)

        @pl.loop(0, n_groups, step=_NBUF)
        def _group(g):
            for p in range(_NBUF):
                grp = g + p
                q = (p + 2) % _NBUF
                drain_gather(p)
                fire_store(grp, p)

                @pl.when(grp >= 2)
                def _wait_prev_store():
                    drain_store(q)

                @pl.when(grp + 2 < n_groups)
                def _refill():
                    fire(grp + 2, q)

        drain_store((n_groups - 2) % _NBUF)
        drain_store((n_groups - 1) % _NBUF)

    return gather


@jax.jit
def kernel(x, word_vectors, W_proj):
    b, s = x.shape
    proj_table = _project_table(word_vectors, W_proj)
    flat_idx = x.reshape(-1).astype(jnp.int32)
    idx2d = flat_idx.reshape(-1, _CHUNK)
    out = _make_gather(b * s, HIDDEN)(proj_table, idx2d)
    return out.reshape(b, s, HIDDEN)


# speedup vs baseline: 1.1388x; 1.0169x over previous
import functools

import jax
import jax.numpy as jnp
from jax import lax
from jax.experimental import pallas as pl
from jax.experimental.pallas import tpu as pltpu
from jax.experimental.pallas import tpu_sc as plsc

VOCAB = 100000
EMBD = 128
HIDDEN = 128

_INFO = plsc.get_sparse_core_info()
_NC, _NS = _INFO.num_cores, _INFO.num_subcores
_NW = _NC * _NS

_CHUNK = 128


def _proj_kernel(wv_ref, w_ref, out_ref):
    out_ref[...] = lax.dot_general(
        wv_ref[...], w_ref[...],
        dimension_numbers=(((1,), (1,)), ((), ())),
        preferred_element_type=jnp.float32,
    )


def _project_table(word_vectors, W_proj):
    rows_per_block = 10000
    grid = VOCAB // rows_per_block
    return pl.pallas_call(
        _proj_kernel,
        grid=(grid,),
        in_specs=[
            pl.BlockSpec((rows_per_block, EMBD), lambda i: (i, 0)),
            pl.BlockSpec((HIDDEN, EMBD), lambda i: (0, 0)),
        ],
        out_specs=pl.BlockSpec((rows_per_block, HIDDEN), lambda i: (i, 0)),
        out_shape=jax.ShapeDtypeStruct((VOCAB, HIDDEN), jnp.float32),
    )(word_vectors, W_proj)


_NBUF = 4


def _make_gather(total, d):
    assert total % (_NW * _CHUNK) == 0
    per_worker = total // _NW
    n_groups = per_worker // _CHUNK
    assert n_groups % _NBUF == 0 and n_groups >= 2 * _NBUF
    mesh = plsc.VectorSubcoreMesh(core_axis_name="c", subcore_axis_name="s")

    @functools.partial(
        pl.kernel,
        mesh=mesh,
        out_type=jax.ShapeDtypeStruct((total, d), jnp.float32),
        scratch_types=[
            pltpu.VMEM((n_groups, _CHUNK), jnp.int32),
        ]
        + [pltpu.VMEM((_CHUNK, d), jnp.float32)] * _NBUF
        + [pltpu.SemaphoreType.DMA] * (2 * _NBUF),
    )
    def gather(table_hbm, idx2d_hbm, out_hbm, idx_all,
               b0, b1, b2, b3, g0, g1, g2, g3, s0, s1, s2, s3):
        wid = lax.axis_index("s") * _NC + lax.axis_index("c")
        base = wid * per_worker
        bufs = (b0, b1, b2, b3)
        gsems = (g0, g1, g2, g3)
        ssems = (s0, s1, s2, s3)

        pltpu.sync_copy(idx2d_hbm.at[pl.ds(wid * n_groups, n_groups)], idx_all)

        def fire(grp, slot):
            pltpu.async_copy(
                table_hbm.at[idx_all.at[grp]], bufs[slot], gsems[slot])

        def drain_gather(slot):
            pltpu.make_async_copy(
                out_hbm.at[pl.ds(0, _CHUNK)], bufs[slot], gsems[slot]).wait()

        def fire_store(grp, slot):
            pltpu.async_copy(
                bufs[slot], out_hbm.at[pl.ds(base + grp * _CHUNK, _CHUNK)],
                ssems[slot])

        def drain_store(slot):
            pltpu.make_async_copy(
                bufs[slot], out_hbm.at[pl.ds(0, _CHUNK)], ssems[slot]).wait()

        fire(0, 0)
        fire(1, 1)

        @pl.loop(0, n_groups, step=_NBUF)
        def _group(g):
            for p in range(_NBUF):
                grp = g + p
                q = (p + 2) % _NBUF
                drain_gather(p)
                fire_store(grp, p)

                @pl.when(grp >= 2)
                def _wait_prev_store():
                    drain_store(q)

                @pl.when(grp + 2 < n_groups)
                def _refill():
                    fire(grp + 2, q)

        drain_store((n_groups - 2) % _NBUF)
        drain_store((n_groups - 1) % _NBUF)

    return gather


@jax.jit
def kernel(x, word_vectors, W_proj):
    b, s = x.shape
    proj_table = _project_table(word_vectors, W_proj)
    flat_idx = x.reshape(-1).astype(jnp.int32)
    idx2d = flat_idx.reshape(-1, _CHUNK)
    out = _make_gather(b * s, HIDDEN)(proj_table, idx2d)
    return out.reshape(b, s, HIDDEN)


# speedup vs baseline: 1.1415x; 1.0024x over previous
import functools

import jax
import jax.numpy as jnp
from jax import lax
from jax.experimental import pallas as pl
from jax.experimental.pallas import tpu as pltpu
from jax.experimental.pallas import tpu_sc as plsc

VOCAB = 100000
EMBD = 128
HIDDEN = 128

_INFO = plsc.get_sparse_core_info()
_NC, _NS = _INFO.num_cores, _INFO.num_subcores
_NW = _NC * _NS

_CHUNK = 128


def _proj_kernel(wv_ref, w_ref, out_ref):
    out_ref[...] = lax.dot_general(
        wv_ref[...], w_ref[...],
        dimension_numbers=(((1,), (1,)), ((), ())),
        preferred_element_type=jnp.float32,
    )


def _project_table(word_vectors, W_proj):
    rows_per_block = 10000
    grid = VOCAB // rows_per_block
    return pl.pallas_call(
        _proj_kernel,
        grid=(grid,),
        in_specs=[
            pl.BlockSpec((rows_per_block, EMBD), lambda i: (i, 0)),
            pl.BlockSpec((HIDDEN, EMBD), lambda i: (0, 0)),
        ],
        out_specs=pl.BlockSpec((rows_per_block, HIDDEN), lambda i: (i, 0)),
        out_shape=jax.ShapeDtypeStruct((VOCAB, HIDDEN), jnp.float32),
    )(word_vectors, W_proj)


_NBUF = 5
_AHEAD = 3


def _make_gather(total, d):
    assert total % (_NW * _CHUNK) == 0
    per_worker = total // _NW
    n_groups = per_worker // _CHUNK
    assert n_groups % _NBUF == 0 and n_groups >= 2 * _NBUF
    mesh = plsc.VectorSubcoreMesh(core_axis_name="c", subcore_axis_name="s")

    @functools.partial(
        pl.kernel,
        mesh=mesh,
        out_type=jax.ShapeDtypeStruct((total, d), jnp.float32),
        scratch_types=[
            pltpu.VMEM((n_groups, _CHUNK), jnp.int32),
        ]
        + [pltpu.VMEM((_CHUNK, d), jnp.float32)] * _NBUF
        + [pltpu.SemaphoreType.DMA] * (2 * _NBUF),
    )
    def gather(table_hbm, idx2d_hbm, out_hbm, idx_all,
               b0, b1, b2, b3, b4, g0, g1, g2, g3, g4, s0, s1, s2, s3, s4):
        wid = lax.axis_index("s") * _NC + lax.axis_index("c")
        base = wid * per_worker
        bufs = (b0, b1, b2, b3, b4)
        gsems = (g0, g1, g2, g3, g4)
        ssems = (s0, s1, s2, s3, s4)

        pltpu.sync_copy(idx2d_hbm.at[pl.ds(wid * n_groups, n_groups)], idx_all)

        def fire(grp, slot):
            pltpu.async_copy(
                table_hbm.at[idx_all.at[grp]], bufs[slot], gsems[slot])

        def drain_gather(slot):
            pltpu.make_async_copy(
                out_hbm.at[pl.ds(0, _CHUNK)], bufs[slot], gsems[slot]).wait()

        def fire_store(grp, slot):
            pltpu.async_copy(
                bufs[slot], out_hbm.at[pl.ds(base + grp * _CHUNK, _CHUNK)],
                ssems[slot])

        def drain_store(slot):
            pltpu.make_async_copy(
                bufs[slot], out_hbm.at[pl.ds(0, _CHUNK)], ssems[slot]).wait()

        for p in range(_AHEAD):
            fire(p, p)

        @pl.loop(0, n_groups, step=_NBUF)
        def _group(g):
            for p in range(_NBUF):
                grp = g + p
                q = (p + _AHEAD) % _NBUF
                drain_gather(p)
                fire_store(grp, p)

                @pl.when(grp >= 2)
                def _wait_prev_store():
                    drain_store(q)

                @pl.when(grp + _AHEAD < n_groups)
                def _refill():
                    fire(grp + _AHEAD, q)

        drain_store((n_groups - 2) % _NBUF)
        drain_store((n_groups - 1) % _NBUF)

    return gather


@jax.jit
def kernel(x, word_vectors, W_proj):
    b, s = x.shape
    proj_table = _project_table(word_vectors, W_proj)
    flat_idx = x.reshape(-1).astype(jnp.int32)
    idx2d = flat_idx.reshape(-1, _CHUNK)
    out = _make_gather(b * s, HIDDEN)(proj_table, idx2d)
    return out.reshape(b, s, HIDDEN)


# speedup vs baseline: 1.1445x; 1.0026x over previous
import functools

import jax
import jax.numpy as jnp
from jax import lax
from jax.experimental import pallas as pl
from jax.experimental.pallas import tpu as pltpu
from jax.experimental.pallas import tpu_sc as plsc

VOCAB = 100000
EMBD = 128
HIDDEN = 128

_INFO = plsc.get_sparse_core_info()
_NC, _NS = _INFO.num_cores, _INFO.num_subcores
_NW = _NC * _NS

_CHUNK = 64


def _proj_kernel(wv_ref, w_ref, out_ref):
    out_ref[...] = lax.dot_general(
        wv_ref[...], w_ref[...],
        dimension_numbers=(((1,), (1,)), ((), ())),
        preferred_element_type=jnp.float32,
    )


def _project_table(word_vectors, W_proj):
    rows_per_block = 10000
    grid = VOCAB // rows_per_block
    return pl.pallas_call(
        _proj_kernel,
        grid=(grid,),
        in_specs=[
            pl.BlockSpec((rows_per_block, EMBD), lambda i: (i, 0)),
            pl.BlockSpec((HIDDEN, EMBD), lambda i: (0, 0)),
        ],
        out_specs=pl.BlockSpec((rows_per_block, HIDDEN), lambda i: (i, 0)),
        out_shape=jax.ShapeDtypeStruct((VOCAB, HIDDEN), jnp.float32),
    )(word_vectors, W_proj)


_NBUF = 5
_AHEAD = 3


def _make_gather(total, d):
    assert total % (_NW * _CHUNK) == 0
    per_worker = total // _NW
    n_groups = per_worker // _CHUNK
    assert n_groups % _NBUF == 0 and n_groups >= 2 * _NBUF
    mesh = plsc.VectorSubcoreMesh(core_axis_name="c", subcore_axis_name="s")

    @functools.partial(
        pl.kernel,
        mesh=mesh,
        out_type=jax.ShapeDtypeStruct((total, d), jnp.float32),
        scratch_types=[
            pltpu.VMEM((n_groups, _CHUNK), jnp.int32),
        ]
        + [pltpu.VMEM((_CHUNK, d), jnp.float32)] * _NBUF
        + [pltpu.SemaphoreType.DMA] * (2 * _NBUF),
    )
    def gather(table_hbm, idx2d_hbm, out_hbm, idx_all,
               b0, b1, b2, b3, b4, g0, g1, g2, g3, g4, s0, s1, s2, s3, s4):
        wid = lax.axis_index("s") * _NC + lax.axis_index("c")
        base = wid * per_worker
        bufs = (b0, b1, b2, b3, b4)
        gsems = (g0, g1, g2, g3, g4)
        ssems = (s0, s1, s2, s3, s4)

        pltpu.sync_copy(idx2d_hbm.at[pl.ds(wid * n_groups, n_groups)], idx_all)

        def fire(grp, slot):
            pltpu.async_copy(
                table_hbm.at[idx_all.at[grp]], bufs[slot], gsems[slot])

        def drain_gather(slot):
            pltpu.make_async_copy(
                out_hbm.at[pl.ds(0, _CHUNK)], bufs[slot], gsems[slot]).wait()

        def fire_store(grp, slot):
            pltpu.async_copy(
                bufs[slot], out_hbm.at[pl.ds(base + grp * _CHUNK, _CHUNK)],
                ssems[slot])

        def drain_store(slot):
            pltpu.make_async_copy(
                bufs[slot], out_hbm.at[pl.ds(0, _CHUNK)], ssems[slot]).wait()

        for p in range(_AHEAD):
            fire(p, p)

        @pl.loop(0, n_groups, step=_NBUF)
        def _group(g):
            for p in range(_NBUF):
                grp = g + p
                q = (p + _AHEAD) % _NBUF
                drain_gather(p)
                fire_store(grp, p)

                @pl.when(grp >= 2)
                def _wait_prev_store():
                    drain_store(q)

                @pl.when(grp + _AHEAD < n_groups)
                def _refill():
                    fire(grp + _AHEAD, q)

        drain_store((n_groups - 2) % _NBUF)
        drain_store((n_groups - 1) % _NBUF)

    return gather


@jax.jit
def kernel(x, word_vectors, W_proj):
    b, s = x.shape
    proj_table = _project_table(word_vectors, W_proj)
    flat_idx = x.reshape(-1).astype(jnp.int32)
    idx2d = flat_idx.reshape(-1, _CHUNK)
    out = _make_gather(b * s, HIDDEN)(proj_table, idx2d)
    return out.reshape(b, s, HIDDEN)
